# baseline (speedup 1.0000x reference)
import functools

import jax
import jax.numpy as jnp
from jax import lax
from jax.experimental import pallas as pl
from jax.experimental.pallas import tpu as pltpu
from jax.experimental.pallas import tpu_sc as plsc

HIDDEN = 1024
NC = 2
NS = 16
NW = NC * NS
CHUNK = 16
B_PER_W = 1024
NCHUNK = B_PER_W // CHUNK
DEPTH = 6
GLEAD = DEPTH // 2


def _gather_body(table_hbm, idx_hbm, out_hbm, idx_v, *scratch):
    rows = scratch[:DEPTH]
    gsem = scratch[DEPTH:2 * DEPTH]
    ssem = scratch[2 * DEPTH:3 * DEPTH]
    wid = lax.axis_index("s") * NC + lax.axis_index("c")
    base = wid * B_PER_W
    pltpu.sync_copy(idx_hbm.at[wid], idx_v)

    def gather(c, b):
        pltpu.async_copy(table_hbm.at[idx_v.at[c]], rows[b], gsem[b])

    def gwait(b):
        pltpu.make_async_copy(table_hbm.at[idx_v.at[0]], rows[b], gsem[b]).wait()

    def scatter(c, b):
        pltpu.async_copy(rows[b], out_hbm.at[pl.ds(base + c * CHUNK, CHUNK)],
                         ssem[b])

    def swait(b):
        pltpu.make_async_copy(rows[b], out_hbm.at[pl.ds(base, CHUNK)],
                              ssem[b]).wait()

    def do_chunk(c, b_cur, b_nxt, first=False, guard=False):
        gwait(b_cur)
        if not first:
            swait(b_nxt)
        if guard:
            @pl.when(c + GLEAD < NCHUNK)
            def _():
                gather(c + GLEAD, b_nxt)
        else:
            gather(c + GLEAD, b_nxt)
        scatter(c, b_cur)

    for c in range(GLEAD):
        gather(c, c % DEPTH)
    for c in range(GLEAD):
        do_chunk(c, c % DEPTH, (c + GLEAD) % DEPTH, first=True)

    n_steady = NCHUNK - GLEAD
    n_groups = (n_steady // DEPTH) - 1

    def step(i, carry):
        c0 = DEPTH * i + GLEAD
        for j in range(DEPTH):
            do_chunk(c0 + j, (GLEAD + j) % DEPTH, j % DEPTH)
        return carry

    lax.fori_loop(0, n_groups, step, 0)

    tail0 = n_groups * DEPTH + GLEAD
    for c in range(tail0, NCHUNK):
        do_chunk(c, c % DEPTH, (c + GLEAD) % DEPTH, guard=True)
    for c in range(NCHUNK - GLEAD, NCHUNK):
        swait(c % DEPTH)


@jax.jit
def _lookup(pe, idx3):
    mesh = plsc.VectorSubcoreMesh(core_axis_name="c", subcore_axis_name="s")
    fn = functools.partial(
        pl.kernel,
        mesh=mesh,
        out_type=jax.ShapeDtypeStruct((NW * B_PER_W, HIDDEN), jnp.float32),
        scratch_types=(
            [pltpu.VMEM((NCHUNK, CHUNK), jnp.int32)]
            + [pltpu.VMEM((CHUNK, HIDDEN), jnp.float32)] * DEPTH
            + [pltpu.SemaphoreType.DMA] * (2 * DEPTH)
        ),
    )(_gather_body)
    return fn(pe, idx3)


def kernel(idx, pe):
    b, s = idx.shape
    idx3 = idx.astype(jnp.int32).reshape(NW, NCHUNK, CHUNK)
    out = _lookup(pe, idx3)
    return out.reshape(b, s, HIDDEN)


# speedup vs baseline: 1.0033x; 1.0033x over previous
import functools

import jax
import jax.numpy as jnp
from jax import lax
from jax.experimental import pallas as pl
from jax.experimental.pallas import tpu as pltpu
from jax.experimental.pallas import tpu_sc as plsc

HIDDEN = 1024
NC = 2
NS = 16
NW = NC * NS
CHUNK = 32
B_PER_W = 1024
NCHUNK = B_PER_W // CHUNK


def _gather_body(table_hbm, idx_hbm, out_hbm,
                 idx_v, rows_0, rows_1, rows_2,
                 gsem_0, gsem_1, gsem_2, ssem_0, ssem_1, ssem_2):
    wid = lax.axis_index("s") * NC + lax.axis_index("c")
    base = wid * B_PER_W
    rows = (rows_0, rows_1, rows_2)
    gsem = (gsem_0, gsem_1, gsem_2)
    ssem = (ssem_0, ssem_1, ssem_2)
    pltpu.sync_copy(idx_hbm.at[wid], idx_v)

    def gather(c, b):
        pltpu.async_copy(table_hbm.at[idx_v.at[c]], rows[b], gsem[b])

    def gwait(b):
        pltpu.make_async_copy(table_hbm.at[idx_v.at[0]], rows[b], gsem[b]).wait()

    def scatter(c, b):
        pltpu.async_copy(rows[b], out_hbm.at[pl.ds(base + c * CHUNK, CHUNK)],
                         ssem[b])

    def swait(b):
        pltpu.make_async_copy(rows[b], out_hbm.at[pl.ds(base, CHUNK)],
                              ssem[b]).wait()

    def do_chunk(c, b_cur, b_n2):
        gwait(b_cur)
        swait(b_n2)

        @pl.when(c + 2 < NCHUNK)
        def _():
            gather(c + 2, b_n2)

        scatter(c, b_cur)

    gather(0, 0)
    gather(1, 1)
    gwait(0)
    gather(2, 2)
    scatter(0, 0)

    def step(i, carry):
        c = 3 * i + 1
        do_chunk(c, 1, 0)
        do_chunk(c + 1, 2, 1)
        do_chunk(c + 2, 0, 2)
        return carry

    lax.fori_loop(0, (NCHUNK - 2) // 3, step, 0)

    c_last = NCHUNK - 1
    gwait(c_last % 3)
    swait((c_last + 2) % 3)
    scatter(c_last, c_last % 3)
    swait(c_last % 3)


@jax.jit
def _lookup(pe, idx3):
    mesh = plsc.VectorSubcoreMesh(core_axis_name="c", subcore_axis_name="s")
    fn = functools.partial(
        pl.kernel,
        mesh=mesh,
        out_type=jax.ShapeDtypeStruct((NW * B_PER_W, HIDDEN), jnp.float32),
        scratch_types=[
            pltpu.VMEM((NCHUNK, CHUNK), jnp.int32),
            pltpu.VMEM((CHUNK, HIDDEN), jnp.float32),
            pltpu.VMEM((CHUNK, HIDDEN), jnp.float32),
            pltpu.VMEM((CHUNK, HIDDEN), jnp.float32),
            pltpu.SemaphoreType.DMA,
            pltpu.SemaphoreType.DMA,
            pltpu.SemaphoreType.DMA,
            pltpu.SemaphoreType.DMA,
            pltpu.SemaphoreType.DMA,
            pltpu.SemaphoreType.DMA,
        ],
    )(_gather_body)
    return fn(pe, idx3)


def kernel(idx, pe):
    b, s = idx.shape
    idx3 = idx.astype(jnp.int32).reshape(NW, NCHUNK, CHUNK)
    out = _lookup(pe, idx3)
    return out.reshape(b, s, HIDDEN)
